# baseline (speedup 1.0000x reference)
import jax
import jax.numpy as jnp
from jax import lax
from jax.experimental import pallas as pl
from jax.experimental.pallas import tpu as pltpu
from jax.experimental.pallas import tpu_sc as plsc

NC = 2
NS = 16
L = 16
NW = NC * NS

K = 32


def _emb_kernel(x_hbm, tab_hbm, pos_hbm, out_hbm, idx_v, buf_a, buf_b, sem):
    n, d = out_hbm.shape
    seq = pos_hbm.shape[0]
    per_w = n // NW
    chunks = per_w // K

    wid = lax.axis_index("s") * NC + lax.axis_index("c")
    base = wid * per_w
    pos0 = lax.rem(base, seq)

    pltpu.sync_copy(x_hbm.at[pl.ds(base, per_w)], idx_v)

    def chunk_body(c, carry):
        off = c * K
        pltpu.sync_copy(pos_hbm.at[pl.ds(pos0 + off, K)], buf_b)
        pltpu.async_copy(tab_hbm.at[idx_v.at[pl.ds(off, K)]], buf_a, sem).wait()

        def row_body(r, carry2):
            for j in range(d // L):
                o = j * L
                v = buf_a[r, pl.ds(o, L)]
                plsc.addupdate(buf_b.at[r, pl.ds(o, L)], v)
            return carry2

        lax.fori_loop(0, K, row_body, 0)
        pltpu.sync_copy(buf_b, out_hbm.at[pl.ds(base + off, K)])
        return carry

    lax.fori_loop(0, chunks, chunk_body, 0)


def kernel(x, token_table, pos_table):
    b, s = x.shape
    v, d = token_table.shape
    n = b * s
    x_flat = x.reshape(n).astype(jnp.int32)

    mesh = plsc.VectorSubcoreMesh(core_axis_name="c", subcore_axis_name="s",
                                  num_cores=NC, num_subcores=NS)
    out = pl.kernel(
        _emb_kernel,
        out_type=jax.ShapeDtypeStruct((n, d), jnp.float32),
        mesh=mesh,
        scratch_types=[
            pltpu.VMEM((n // NW,), jnp.int32),
            pltpu.VMEM((K, d), jnp.float32),
            pltpu.VMEM((K, d), jnp.float32),
            pltpu.SemaphoreType.DMA,
        ],
    )(x_flat, token_table, pos_table)
    return out.reshape(b, s, d)


# speedup vs baseline: 1.3814x; 1.3814x over previous
import jax
import jax.numpy as jnp
from jax import lax
from jax.experimental import pallas as pl
from jax.experimental.pallas import tpu as pltpu
from jax.experimental.pallas import tpu_sc as plsc

NC = 2
NS = 16
L = 16
NW = NC * NS

K = 16


def _emb_kernel(x_hbm, tab_hbm, pos_hbm, out_hbm, idx_v,
                a0, a1, p0, p1, o0, o1,
                sg0, sg1, sp0, sp1, so0, so1):
    n, d = out_hbm.shape
    seq = pos_hbm.shape[0]
    per_w = n // NW
    chunks = per_w // K
    rounds = chunks // 2

    a = (a0, a1)
    p = (p0, p1)
    o = (o0, o1)
    sg = (sg0, sg1)
    sp = (sp0, sp1)
    so = (so0, so1)

    wid = lax.axis_index("s") * NC + lax.axis_index("c")
    base = wid * per_w
    pos0 = lax.rem(base, seq)

    pltpu.sync_copy(x_hbm.at[pl.ds(base, per_w)], idx_v)

    def start_gather(c, b):
        pltpu.async_copy(tab_hbm.at[idx_v.at[pl.ds(c * K, K)]], a[b], sg[b])

    def start_pos(c, b):
        pltpu.async_copy(pos_hbm.at[pl.ds(pos0 + c * K, K)], p[b], sp[b])

    def start_out(c, b):
        pltpu.async_copy(o[b], out_hbm.at[pl.ds(base + c * K, K)], so[b])

    def wait_gather(c, b):
        pltpu.make_async_copy(tab_hbm.at[idx_v.at[pl.ds(c * K, K)]],
                              a[b], sg[b]).wait()

    def wait_pos(c, b):
        pltpu.make_async_copy(pos_hbm.at[pl.ds(pos0 + c * K, K)],
                              p[b], sp[b]).wait()

    def wait_out(c, b):
        pltpu.make_async_copy(o[b], out_hbm.at[pl.ds(base + c * K, K)],
                              so[b]).wait()

    def compute(b):
        def row_body(r, carry):
            for j in range(d // L):
                sl = pl.ds(j * L, L)
                o[b][r, sl] = a[b][r, sl] + p[b][r, sl]
            return carry
        lax.fori_loop(0, K, row_body, 0)

    for b in range(2):
        start_gather(b, b)
        start_pos(b, b)

    def round_body(r, carry):
        for b in range(2):
            c = 2 * r + b
            wait_gather(c, b)
            wait_pos(c, b)

            @pl.when(r >= 1)
            def _():
                wait_out(c - 2, b)

            compute(b)
            start_out(c, b)
            start_gather(c + 2, b)
            start_pos(c + 2, b)
        return carry

    lax.fori_loop(0, rounds - 1, round_body, 0)

    for b in range(2):
        c = chunks - 2 + b
        wait_gather(c, b)
        wait_pos(c, b)
        wait_out(c - 2, b)
        compute(b)
        start_out(c, b)
    for b in range(2):
        wait_out(chunks - 2 + b, b)


def kernel(x, token_table, pos_table):
    b, s = x.shape
    v, d = token_table.shape
    n = b * s
    x_flat = x.reshape(n).astype(jnp.int32)

    mesh = plsc.VectorSubcoreMesh(core_axis_name="c", subcore_axis_name="s",
                                  num_cores=NC, num_subcores=NS)
    out = pl.kernel(
        _emb_kernel,
        out_type=jax.ShapeDtypeStruct((n, d), jnp.float32),
        mesh=mesh,
        scratch_types=[
            pltpu.VMEM((n // NW,), jnp.int32),
            pltpu.VMEM((K, d), jnp.float32),
            pltpu.VMEM((K, d), jnp.float32),
            pltpu.VMEM((K, d), jnp.float32),
            pltpu.VMEM((K, d), jnp.float32),
            pltpu.VMEM((K, d), jnp.float32),
            pltpu.VMEM((K, d), jnp.float32),
            pltpu.SemaphoreType.DMA,
            pltpu.SemaphoreType.DMA,
            pltpu.SemaphoreType.DMA,
            pltpu.SemaphoreType.DMA,
            pltpu.SemaphoreType.DMA,
            pltpu.SemaphoreType.DMA,
        ],
    )(x_flat, token_table, pos_table)
    return out.reshape(b, s, d)
